# baseline (speedup 1.0000x reference)
import jax
import jax.numpy as jnp
from jax.experimental import pallas as pl
from jax.experimental.pallas import tpu as pltpu

N = 1024
C = 1024
KV = 128
KT = 128
KSEL = KV + KT
K2 = int((N - KSEL) * 0.25)
NOUT = KSEL + K2
SCALE = C ** -0.5
T = 4
TR = N // T


def _body(ar_ref, ac_ref, sr_ref, sc_ref, key_hbm, img_hbm, out_hbm,
          key_v, img_v, key_b, out_v, q_scr, in_sem, out_sem):
    f32 = jnp.float32
    bf16 = jnp.bfloat16

    key_cp = [pltpu.make_async_copy(key_hbm.at[t * TR:(t + 1) * TR, :],
                                    key_v.at[t * TR:(t + 1) * TR, :],
                                    in_sem.at[t]) for t in range(T)]
    img_cp = [pltpu.make_async_copy(img_hbm.at[t * TR:(t + 1) * TR, :],
                                    img_v.at[t * TR:(t + 1) * TR, :],
                                    in_sem.at[T + t]) for t in range(T)]
    for cp in key_cp:
        cp.start()
    for cp in img_cp:
        cp.start()

    ar = ar_ref[...]
    ac = ac_ref[...]
    sr = sr_ref[...]
    sc = sc_ref[...]

    ioj = jax.lax.broadcasted_iota(jnp.int32, (N, N), 0)
    ioi = jax.lax.broadcasted_iota(jnp.int32, (N, N), 1)
    ident = (ioj == ioi).astype(f32)

    def to_col(vr):
        return jax.lax.dot_general(ident, vr, (((1,), (1,)), ((), ())),
                                   preferred_element_type=f32)

    def rank_row(vc, vr):
        m = (vc > vr) | ((vc == vr) & (ioj < ioi))
        return jnp.sum(m.astype(f32), axis=0, keepdims=True)

    rv_r = rank_row(ac, ar)
    rt_r = rank_row(sc, sr)
    sel_r = ((rv_r < KV) | (rt_r < KT)).astype(f32)
    k1_r = sel_r * 1e6 + ar
    k1_c = to_col(k1_r)
    rs_r = rank_row(k1_c, k1_r)
    a_r = rs_r < KSEL
    a_c = to_col(a_r.astype(f32)) > 0.5

    m2 = (~a_c) & ((ac > ar) | ((ac == ar) & (ioj < ioi)))
    r2_r = jnp.sum(m2.astype(f32), axis=0, keepdims=True)
    b_r = (~a_r) & (r2_r < K2)
    cmask = (~a_r) & (~b_r)

    row_of = jnp.where(a_r, rs_r, jnp.where(b_r, KSEL + r2_r, 2.0 * N))
    io_out = jax.lax.broadcasted_iota(jnp.int32, (NOUT, N), 0)
    q_scr[0:NOUT, :] = (io_out == row_of.astype(jnp.int32)).astype(bf16)

    neg = jnp.float32(-jnp.inf)
    t_log = jnp.where(cmask, 50.0 * sr, neg)
    te = jnp.exp(t_log - jnp.max(t_log, axis=1, keepdims=True))
    sm = te / jnp.sum(te, axis=1, keepdims=True)
    score = ar * sm

    kb = jnp.zeros((K2, C), f32)
    nk2_parts = []
    for t in range(T):
        key_cp[t].wait()
        key_t = key_v[t * TR:(t + 1) * TR, :]
        key_b[t * TR:(t + 1) * TR, :] = key_t.astype(bf16)
        kb += jax.lax.dot_general(q_scr[KSEL:NOUT, t * TR:(t + 1) * TR],
                                  key_b[t * TR:(t + 1) * TR, :],
                                  (((1,), (0,)), ((), ())),
                                  preferred_element_type=f32)
        nk2_parts.append(jnp.sum(key_t * key_t, axis=1, keepdims=True))
    nk2 = jnp.concatenate(nk2_parts, axis=0)

    kb_b = kb.astype(bf16)
    cos_parts = []
    for t in range(T):
        cos_parts.append(jax.lax.dot_general(kb_b, key_b[t * TR:(t + 1) * TR, :],
                                             (((1,), (1,)), ((), ())),
                                             preferred_element_type=f32))
    cos = jnp.concatenate(cos_parts, axis=1)

    nk2_r = jax.lax.dot_general(nk2, ident, (((0,), (0,)), ((), ())),
                                preferred_element_type=f32)
    nb = jnp.maximum(jnp.sqrt(jnp.sum(kb * kb, axis=1, keepdims=True)),
                     1e-12)
    nk = jnp.maximum(jnp.sqrt(nk2_r), 1e-12)
    cosn = cos / nb / nk * SCALE
    logits = jnp.where(cmask, cosn, neg)
    e = jnp.exp(logits - jnp.max(logits, axis=1, keepdims=True))
    w = e / jnp.sum(e, axis=1, keepdims=True)
    q_scr[NOUT:NOUT + K2, :] = (w * score).astype(bf16)

    ym = jnp.zeros((KSEL, C), f32)
    yr = jnp.zeros((2 * K2, C), f32)
    for t in range(T):
        img_cp[t].wait()
        img_t = img_v[t * TR:(t + 1) * TR, :].astype(bf16)
        ym += jax.lax.dot_general(q_scr[0:KSEL, t * TR:(t + 1) * TR], img_t,
                                  (((1,), (0,)), ((), ())),
                                  preferred_element_type=f32)
        yr += jax.lax.dot_general(q_scr[KSEL:NOUT + K2, t * TR:(t + 1) * TR],
                                  img_t, (((1,), (0,)), ((), ())),
                                  preferred_element_type=f32)

    out_v[0:KSEL, :] = ym
    cp0 = pltpu.make_async_copy(out_v.at[0:KSEL, :], out_hbm.at[0:KSEL, :],
                                out_sem.at[0])
    cp0.start()
    out_v[KSEL:NOUT, :] = yr[0:K2, :] + yr[K2:2 * K2, :]
    cp1 = pltpu.make_async_copy(out_v.at[KSEL:NOUT, :],
                                out_hbm.at[KSEL:NOUT, :], out_sem.at[1])
    cp1.start()
    cp0.wait()
    cp1.wait()


def kernel(image_features, key_features, cls_attn, similarity):
    img = image_features[0]
    key = key_features[0]
    ar = cls_attn
    ac = cls_attn.reshape(N, 1)
    sr = similarity
    sc = similarity.reshape(N, 1)
    f32 = jnp.float32
    out = pl.pallas_call(
        _body,
        in_specs=[
            pl.BlockSpec((1, N), lambda: (0, 0)),
            pl.BlockSpec((N, 1), lambda: (0, 0)),
            pl.BlockSpec((1, N), lambda: (0, 0)),
            pl.BlockSpec((N, 1), lambda: (0, 0)),
            pl.BlockSpec(memory_space=pl.ANY),
            pl.BlockSpec(memory_space=pl.ANY),
        ],
        out_specs=pl.BlockSpec(memory_space=pl.ANY),
        out_shape=jax.ShapeDtypeStruct((NOUT, C), f32),
        scratch_shapes=[
            pltpu.VMEM((N, C), f32),
            pltpu.VMEM((N, C), f32),
            pltpu.VMEM((N, C), jnp.bfloat16),
            pltpu.VMEM((NOUT, C), f32),
            pltpu.VMEM((NOUT + K2, N), jnp.bfloat16),
            pltpu.SemaphoreType.DMA((2 * T,)),
            pltpu.SemaphoreType.DMA((2,)),
        ],
    )(ar, ac, sr, sc, key, img)
    return out[None]


# speedup vs baseline: 1.4934x; 1.4934x over previous
import jax
import jax.numpy as jnp
from jax.experimental import pallas as pl
from jax.experimental.pallas import tpu as pltpu

N = 1024
C = 1024
KV = 128
KT = 128
KSEL = KV + KT
K2 = int((N - KSEL) * 0.25)
NOUT = KSEL + K2
SCALE = C ** -0.5
T = 4
TR = N // T


def _body(ar_ref, sr_ref, key_hbm, img_hbm, out_hbm,
          key_v, img_v, out_v, q_scr, in_sem, out_sem):
    f32 = jnp.float32
    bf16 = jnp.bfloat16

    key_cp = [pltpu.make_async_copy(key_hbm.at[t * TR:(t + 1) * TR, :],
                                    key_v.at[t * TR:(t + 1) * TR, :],
                                    in_sem.at[t]) for t in range(T)]
    img_cp = [pltpu.make_async_copy(img_hbm.at[t * TR:(t + 1) * TR, :],
                                    img_v.at[t * TR:(t + 1) * TR, :],
                                    in_sem.at[T + t]) for t in range(T)]
    for cp in key_cp:
        cp.start()
    for cp in img_cp:
        cp.start()

    ar = ar_ref[...]
    sr = sr_ref[...]

    ioj = jax.lax.broadcasted_iota(jnp.int32, (N, N), 0)
    ioi = jax.lax.broadcasted_iota(jnp.int32, (N, N), 1)
    ident = (ioj == ioi).astype(f32)

    def to_col(vr):
        return jax.lax.dot_general(ident, vr, (((1,), (1,)), ((), ())),
                                   preferred_element_type=f32)

    ac = to_col(ar)
    sc = to_col(sr)

    def rank_row(vc, vr):
        m = (vc > vr) | ((vc == vr) & (ioj < ioi))
        return jnp.sum(m.astype(f32), axis=0, keepdims=True)

    rv_r = rank_row(ac, ar)
    rt_r = rank_row(sc, sr)
    sel_r = ((rv_r < KV) | (rt_r < KT)).astype(f32)
    k1_r = sel_r * 1e6 + ar
    k1_c = to_col(k1_r)
    rs_r = rank_row(k1_c, k1_r)
    a_r = rs_r < KSEL
    a_c = to_col(a_r.astype(f32)) > 0.5

    m2 = (~a_c) & ((ac > ar) | ((ac == ar) & (ioj < ioi)))
    r2_r = jnp.sum(m2.astype(f32), axis=0, keepdims=True)
    b_r = (~a_r) & (r2_r < K2)
    cmask = (~a_r) & (~b_r)

    row_of = jnp.where(a_r, rs_r, jnp.where(b_r, KSEL + r2_r, 2.0 * N))
    io_out = jax.lax.broadcasted_iota(jnp.int32, (NOUT, N), 0)
    q_scr[0:NOUT, :] = (io_out == row_of.astype(jnp.int32)).astype(bf16)

    neg = jnp.float32(-jnp.inf)
    t_log = jnp.where(cmask, 50.0 * sr, neg)
    te = jnp.exp(t_log - jnp.max(t_log, axis=1, keepdims=True))
    sm = te / jnp.sum(te, axis=1, keepdims=True)
    score = ar * sm

    kb = jnp.zeros((K2, C), f32)
    nk2_parts = []
    for t in range(T):
        key_cp[t].wait()
        key_t = key_v[t * TR:(t + 1) * TR, :]
        kb += jax.lax.dot_general(q_scr[KSEL:NOUT, t * TR:(t + 1) * TR],
                                  key_t.astype(bf16),
                                  (((1,), (0,)), ((), ())),
                                  preferred_element_type=f32)
        nk2_parts.append(jnp.sum(key_t * key_t, axis=1, keepdims=True))
    nk2 = jnp.concatenate(nk2_parts, axis=0)

    kb_b = kb.astype(bf16)
    cos_parts = []
    for t in range(T):
        key_t = key_v[t * TR:(t + 1) * TR, :]
        cos_parts.append(jax.lax.dot_general(kb_b, key_t.astype(bf16),
                                             (((1,), (1,)), ((), ())),
                                             preferred_element_type=f32))
    cos = jnp.concatenate(cos_parts, axis=1)

    nk2_r = jax.lax.dot_general(nk2, ident, (((0,), (0,)), ((), ())),
                                preferred_element_type=f32)
    nb = jnp.maximum(jnp.sqrt(jnp.sum(kb * kb, axis=1, keepdims=True)),
                     1e-12)
    nk = jnp.maximum(jnp.sqrt(nk2_r), 1e-12)
    cosn = cos / nb / nk * SCALE
    logits = jnp.where(cmask, cosn, neg)
    e = jnp.exp(logits - jnp.max(logits, axis=1, keepdims=True))
    w = e / jnp.sum(e, axis=1, keepdims=True)
    q_scr[NOUT:NOUT + K2, :] = (w * score).astype(bf16)

    for cp in img_cp:
        cp.wait()
    img = img_v[...].astype(bf16)
    main = jax.lax.dot_general(q_scr[0:KSEL, :], img, (((1,), (0,)), ((), ())),
                               preferred_element_type=f32)
    out_v[0:KSEL, :] = main
    cp0 = pltpu.make_async_copy(out_v.at[0:KSEL, :], out_hbm.at[0:KSEL, :],
                                out_sem.at[0])
    cp0.start()
    rest = jax.lax.dot_general(q_scr[KSEL:NOUT + K2, :], img,
                               (((1,), (0,)), ((), ())),
                               preferred_element_type=f32)
    out_v[KSEL:NOUT, :] = rest[0:K2, :] + rest[K2:2 * K2, :]
    cp1 = pltpu.make_async_copy(out_v.at[KSEL:NOUT, :],
                                out_hbm.at[KSEL:NOUT, :], out_sem.at[1])
    cp1.start()
    cp0.wait()
    cp1.wait()


def kernel(image_features, key_features, cls_attn, similarity):
    img = image_features[0]
    key = key_features[0]
    f32 = jnp.float32
    out = pl.pallas_call(
        _body,
        in_specs=[
            pl.BlockSpec((1, N), lambda: (0, 0)),
            pl.BlockSpec((1, N), lambda: (0, 0)),
            pl.BlockSpec(memory_space=pl.ANY),
            pl.BlockSpec(memory_space=pl.ANY),
        ],
        out_specs=pl.BlockSpec(memory_space=pl.ANY),
        out_shape=jax.ShapeDtypeStruct((NOUT, C), f32),
        scratch_shapes=[
            pltpu.VMEM((N, C), f32),
            pltpu.VMEM((N, C), f32),
            pltpu.VMEM((NOUT, C), f32),
            pltpu.VMEM((NOUT + K2, N), jnp.bfloat16),
            pltpu.SemaphoreType.DMA((2 * T,)),
            pltpu.SemaphoreType.DMA((2,)),
        ],
    )(cls_attn, similarity, key, img)
    return out[None]
